# baseline (speedup 1.0000x reference)
import jax
import jax.numpy as jnp
from jax.experimental import pallas as pl
from jax.experimental.pallas import tpu as pltpu

N, F_IN, NHID = 4096, 512, 256
XCH = 8
XR = N // XCH
OB = 512
NW = N // OB


def _mm(a, b):
    return jax.lax.dot_general(a, b, (((1,), (0,)), ((), ())),
                               preferred_element_type=jnp.float32)


def _mmT(a, b):
    return jax.lax.dot_general(a, b, (((1,), (1,)), ((), ())),
                               preferred_element_type=jnp.float32)


def _body(x_hbm, *refs):
    w_hbm = refs[:26]
    out_hbm = refs[26]
    xs = refs[27]
    w_vmem = refs[28:54]
    bcast = refs[54]
    (sem_x, sem_gin, sem_l0, sem_l1, sem_head, sem_o) = refs[55:61]

    GIN_IDX = list(range(0, 16))
    L0_IDX = [18, 19, 20, 21]
    L1_IDX = [22, 23, 24, 25]
    HEAD_IDX = [16, 17]

    x_copies = [pltpu.make_async_copy(
        x_hbm.at[pl.ds(k * XR, XR), :], xs.at[pl.ds(k * XR, XR), :], sem_x)
        for k in range(XCH)]
    for c in x_copies:
        c.start()

    def group(idx, sem):
        return [pltpu.make_async_copy(w_hbm[j], w_vmem[j], sem) for j in idx]

    gin_copies = group(GIN_IDX, sem_gin)
    l0_copies = group(L0_IDX, sem_l0)
    l1_copies = group(L1_IDX, sem_l1)
    head_copies = group(HEAD_IDX, sem_head)
    for c in gin_copies + l0_copies + l1_copies + head_copies:
        c.start()

    for c in x_copies:
        c.wait()

    s = jnp.sum(xs[...], axis=0, keepdims=True)

    (W1a, b1a, W1b, b1b, W2a, b2a, W2b, b2b,
     W3a, b3a, W3b, b3b, W4a, b4a, W4b, b4b, Wm, bm,
     Wih0, Whh0, bih0, bhh0, Wih1, Whh1, bih1, bhh1) = w_vmem

    relu = jax.nn.relu

    def gin(z, Wa, ba, Wb, bb):
        return relu(_mm(relu(_mm(z, Wa[...]) + ba[...]), Wb[...]) + bb[...])

    for c in gin_copies:
        c.wait()

    r1 = gin(s, W1a, b1a, W1b, b1b)
    r2 = gin(jnp.float32(N) * r1, W2a, b2a, W2b, b2b)
    r3 = gin(jnp.float32(N) * r2, W3a, b3a, W3b, b3b)
    r4 = gin(jnp.float32(N) * r3, W4a, b4a, W4b, b4b)

    x2 = r2 + r1
    x3 = r3 + x2
    x4 = r4 + x3
    xsum = r1 + x2 + x3 + x4
    seq = [xsum, r1, x2, x3, x4]

    x_avg = (xsum + r1 + x2 + x3 + x4) * jnp.float32(0.2)
    x_max = jnp.maximum(jnp.maximum(jnp.maximum(xsum, r1),
                                    jnp.maximum(x2, x3)), x4)

    def lstm(seq_rows, Wih, Whh, bih, bhh):
        xg = _mmT(jnp.concatenate(seq_rows, axis=0), Wih[...]) \
             + bih[...] + bhh[...]
        h = jnp.zeros((1, NHID), jnp.float32)
        c = jnp.zeros((1, NHID), jnp.float32)
        outs = []
        for t in range(5):
            g = xg[t:t + 1, :] + _mmT(h, Whh[...])
            gi = jax.nn.sigmoid(g[:, 0 * NHID:1 * NHID])
            gf = jax.nn.sigmoid(g[:, 1 * NHID:2 * NHID])
            gg = jnp.tanh(g[:, 2 * NHID:3 * NHID])
            go = jax.nn.sigmoid(g[:, 3 * NHID:4 * NHID])
            c = gf * c + gi * gg
            h = go * jnp.tanh(c)
            outs.append(h)
        return outs

    for c in l0_copies:
        c.wait()
    s1 = lstm(seq, Wih0, Whh0, bih0, bhh0)
    for c in l1_copies:
        c.wait()
    s2 = lstm(s1, Wih1, Whh1, bih1, bhh1)
    x_lstm = (s2[0] + s2[1] + s2[2] + s2[3] + s2[4]) * jnp.float32(0.2)

    for c in head_copies:
        c.wait()
    row = relu((_mm(x_avg, Wm[...]) + bm[...])
               + (_mm(x_max, Wm[...]) + bm[...])
               + (_mm(x_lstm, Wm[...]) + bm[...]))

    bcast[...] = jnp.broadcast_to(row, (OB, NHID))
    o_copies = [pltpu.make_async_copy(
        bcast, out_hbm.at[pl.ds(k * OB, OB), :], sem_o)
        for k in range(NW)]
    for c in o_copies:
        c.start()
    for c in o_copies:
        c.wait()


def kernel(x, W1a, b1a, W1b, b1b, W2a, b2a, W2b, b2b, W3a, b3a, W3b, b3b,
           W4a, b4a, W4b, b4b, Wm, bm, Wih0, Whh0, bih0, bhh0,
           Wih1, Whh1, bih1, bhh1):
    weights = [W1a, b1a, W1b, b1b, W2a, b2a, W2b, b2b,
               W3a, b3a, W3b, b3b, W4a, b4a, W4b, b4b, Wm, bm,
               Wih0, Whh0, bih0, bhh0, Wih1, Whh1, bih1, bhh1]
    weights = [w if w.ndim == 2 else w.reshape(1, -1) for w in weights]

    out = pl.pallas_call(
        _body,
        in_specs=[pl.BlockSpec(memory_space=pl.ANY)] * 27,
        out_specs=pl.BlockSpec(memory_space=pl.ANY),
        out_shape=jax.ShapeDtypeStruct((N, NHID), jnp.float32),
        scratch_shapes=[pltpu.VMEM((N, F_IN), jnp.float32)]
        + [pltpu.VMEM(w.shape, jnp.float32) for w in weights]
        + [pltpu.VMEM((OB, NHID), jnp.float32)]
        + [pltpu.SemaphoreType.DMA] * 6,
    )(x, *weights)
    return out


# speedup vs baseline: 1.0236x; 1.0236x over previous
import jax
import jax.numpy as jnp
from jax.experimental import pallas as pl
from jax.experimental.pallas import tpu as pltpu

N, F_IN, NHID = 4096, 512, 256
XCH = 16
XR = N // XCH
OB = 256
NW = N // OB


def _mm(a, b):
    return jax.lax.dot_general(a, b, (((1,), (0,)), ((), ())),
                               preferred_element_type=jnp.float32)


def _mmT(a, b):
    return jax.lax.dot_general(a, b, (((1,), (1,)), ((), ())),
                               preferred_element_type=jnp.float32)


def _body(x_hbm, *refs):
    w_hbm = refs[:26]
    out_hbm = refs[26]
    xs = refs[27]
    w_vmem = refs[28:54]
    bcast = refs[54]
    sem_x, sem_w, sem_o = refs[55], refs[56], refs[57]

    x_copies = [pltpu.make_async_copy(
        x_hbm.at[pl.ds(k * XR, XR), :], xs.at[pl.ds(k * XR, XR), :], sem_x)
        for k in range(XCH)]
    for c in x_copies:
        c.start()
    w_copies = [pltpu.make_async_copy(w_hbm[j], w_vmem[j], sem_w)
                for j in range(26)]
    for c in w_copies:
        c.start()
    for c in x_copies:
        c.wait()

    s = jnp.sum(xs[...], axis=0, keepdims=True)

    for c in w_copies:
        c.wait()

    (W1a, b1a, W1b, b1b, W2a, b2a, W2b, b2b,
     W3a, b3a, W3b, b3b, W4a, b4a, W4b, b4b, Wm, bm,
     Wih0, Whh0, bih0, bhh0, Wih1, Whh1, bih1, bhh1) = w_vmem

    relu = jax.nn.relu

    def gin(z, Wa, ba, Wb, bb):
        return relu(_mm(relu(_mm(z, Wa[...]) + ba[...]), Wb[...]) + bb[...])

    r1 = gin(s, W1a, b1a, W1b, b1b)
    r2 = gin(jnp.float32(N) * r1, W2a, b2a, W2b, b2b)
    r3 = gin(jnp.float32(N) * r2, W3a, b3a, W3b, b3b)
    r4 = gin(jnp.float32(N) * r3, W4a, b4a, W4b, b4b)

    x2 = r2 + r1
    x3 = r3 + x2
    x4 = r4 + x3
    xsum = r1 + x2 + x3 + x4
    seq = [xsum, r1, x2, x3, x4]

    x_avg = (xsum + r1 + x2 + x3 + x4) * jnp.float32(0.2)
    x_max = jnp.maximum(jnp.maximum(jnp.maximum(xsum, r1),
                                    jnp.maximum(x2, x3)), x4)

    def lstm(seq_rows, Wih, Whh, bih, bhh):
        xg = _mmT(jnp.concatenate(seq_rows, axis=0), Wih[...]) \
             + bih[...] + bhh[...]
        h = jnp.zeros((1, NHID), jnp.float32)
        c = jnp.zeros((1, NHID), jnp.float32)
        outs = []
        for t in range(5):
            g = xg[t:t + 1, :] + _mmT(h, Whh[...])
            gi = jax.nn.sigmoid(g[:, 0 * NHID:1 * NHID])
            gf = jax.nn.sigmoid(g[:, 1 * NHID:2 * NHID])
            gg = jnp.tanh(g[:, 2 * NHID:3 * NHID])
            go = jax.nn.sigmoid(g[:, 3 * NHID:4 * NHID])
            c = gf * c + gi * gg
            h = go * jnp.tanh(c)
            outs.append(h)
        return outs

    s1 = lstm(seq, Wih0, Whh0, bih0, bhh0)
    s2 = lstm(s1, Wih1, Whh1, bih1, bhh1)
    x_lstm = (s2[0] + s2[1] + s2[2] + s2[3] + s2[4]) * jnp.float32(0.2)

    row = relu((_mm(x_avg, Wm[...]) + bm[...])
               + (_mm(x_max, Wm[...]) + bm[...])
               + (_mm(x_lstm, Wm[...]) + bm[...]))

    bcast[...] = jnp.broadcast_to(row, (OB, NHID))
    o_copies = [pltpu.make_async_copy(
        bcast, out_hbm.at[pl.ds(k * OB, OB), :], sem_o)
        for k in range(NW)]
    for c in o_copies:
        c.start()
    for c in o_copies:
        c.wait()


def kernel(x, W1a, b1a, W1b, b1b, W2a, b2a, W2b, b2b, W3a, b3a, W3b, b3b,
           W4a, b4a, W4b, b4b, Wm, bm, Wih0, Whh0, bih0, bhh0,
           Wih1, Whh1, bih1, bhh1):
    weights = [W1a, b1a, W1b, b1b, W2a, b2a, W2b, b2b,
               W3a, b3a, W3b, b3b, W4a, b4a, W4b, b4b, Wm, bm,
               Wih0, Whh0, bih0, bhh0, Wih1, Whh1, bih1, bhh1]
    weights = [w if w.ndim == 2 else w.reshape(1, -1) for w in weights]

    out = pl.pallas_call(
        _body,
        in_specs=[pl.BlockSpec(memory_space=pl.ANY)] * 27,
        out_specs=pl.BlockSpec(memory_space=pl.ANY),
        out_shape=jax.ShapeDtypeStruct((N, NHID), jnp.float32),
        scratch_shapes=[pltpu.VMEM((N, F_IN), jnp.float32)]
        + [pltpu.VMEM(w.shape, jnp.float32) for w in weights]
        + [pltpu.VMEM((OB, NHID), jnp.float32),
           pltpu.SemaphoreType.DMA, pltpu.SemaphoreType.DMA,
           pltpu.SemaphoreType.DMA],
    )(x, *weights)
    return out
